# baseline (speedup 1.0000x reference)
import math

import jax
import jax.numpy as jnp
import numpy as np
from jax.experimental import pallas as pl
from jax.experimental.pallas import tpu as pltpu

L = 96
D = 512
H = 8
E = 64
DFF = 2048
NLAYERS = 2
PRED = 96
PT = 12
REG = 6
ROWS = 32
NG = 2
SG = ROWS // NG
PG = SG * PT
P = ROWS * PT
_INV_SQRT2 = 1.0 / math.sqrt(2.0)
_BF = jnp.bfloat16
_F32 = jnp.float32


def _pos_emb_const():
    position = np.arange(PT, dtype=np.float32)[:, None]
    div = np.exp(np.arange(0, D, 2, dtype=np.float32) * -(math.log(10000.0) / D))
    pe = np.zeros((PT, D), dtype=np.float32)
    pe[:, 0::2] = np.sin(position * div)
    pe[:, 1::2] = np.cos(position * div)
    return pe


def _mm(a, b):
    return jax.lax.dot_general(
        a.astype(_BF), b,
        (((1,), (0,)), ((), ())), preferred_element_type=_F32)


def _mm32(a, b):
    return jax.lax.dot_general(
        a, b, (((1,), (0,)), ((), ())), preferred_element_type=_F32)


def _mm_nt(a, b):
    return jax.lax.dot_general(
        a, b, (((1,), (1,)), ((), ())), preferred_element_type=_F32)


def _ln(x, g, b, cmean):
    xb = x.astype(_BF)
    mu = _mm(xb, cmean)
    m2 = _mm(xb * xb, cmean)
    var = m2 - mu * mu
    return (x - mu) * jax.lax.rsqrt(var + 1e-5) * g + b


def _fused_kernel(x_ref, w1t_ref, b1_ref, w2t_ref, b2_ref, e0t_ref, e1t_ref,
                  pe_ref, mask_ref, cmean_ref, blk8_ref, lw_refs, gs_ref,
                  bnb_ref, hwt_ref, hb_ref, out_ref):
    x = x_ref[...]
    mu = jnp.mean(x, axis=1, keepdims=True)
    xc = x - mu
    var = jnp.mean(xc * xc, axis=1, keepdims=True)
    std = jnp.sqrt(var + 1e-5)
    xn = xc / std

    w1t = w1t_ref[...]
    w2t = w2t_ref[...]
    e0t = e0t_ref[...]
    e1t = e1t_ref[...]
    b1 = b1_ref[...]
    b2 = b2_ref[...]
    pe = pe_ref[...]

    tokens = [None] * (NG * PT)
    for r in range(REG):
        reg = xn[:, 16 * r:16 * r + 16]
        h1 = jax.nn.relu(_mm32(reg, w1t) + b1)
        lg = _mm32(h1, w2t) + b2
        sel = (lg[:, 0:1] >= lg[:, 1:2]).astype(_F32)
        e1 = _mm32(reg, e1t)
        for j in range(2):
            p = 2 * r + j
            pat0 = _mm32(xn[:, 8 * p:8 * p + 8], e0t)
            tok = e1 + sel * (pat0 - e1) + pe[p:p + 1, :]
            for g in range(NG):
                tokens[g * PT + p] = tok[g * SG:(g + 1) * SG]
    enc = jnp.concatenate(tokens, axis=0)

    mask = mask_ref[...]
    cmean = cmean_ref[...]
    ones_col = jnp.ones((PG, 1), _BF)
    for lyr in range(NLAYERS):
        (wq, bq, wk, bk, wv, bv, wo, bo,
         c1, c1b, c2, c2b, n1g, n1b, n2g, n2b) = [
            ref[...] for ref in lw_refs[16 * lyr:16 * (lyr + 1)]]
        xb = enc.astype(_BF)
        q = (_mm(xb, wq) + bq).astype(_BF)
        k = (_mm(xb, wk) + bk).astype(_BF)
        v = (_mm(xb, wv) + bv).astype(_BF)
        parts = []
        dens = []
        for g in range(NG):
            r0 = g * PG
            heads = []
            gdens = []
            for h in range(H):
                qh = q[r0:r0 + PG, E * h:E * h + E]
                kh = k[r0:r0 + PG, E * h:E * h + E]
                s = _mm_nt(qh, kh)
                a = jnp.exp(s).astype(_BF) * mask
                vh = jnp.concatenate(
                    [v[r0:r0 + PG, E * h:E * h + E], ones_col], axis=1)
                o = _mm(a, vh)
                heads.append(o[:, :E])
                gdens.append(o[:, E:E + 1])
            parts.append(jnp.concatenate(heads, axis=1))
            dens.append(jnp.concatenate(gdens, axis=1))
        ao = jnp.concatenate(parts, axis=0)
        den = jnp.concatenate(dens, axis=0)
        denrep = _mm((1.0 / den), blk8_ref[...])
        ao = _mm(ao * denrep, wo) + bo
        enc = enc + ao
        enc = _ln(enc, n1g, n1b, cmean)
        y = _mm(enc, c1) + c1b
        yb = y.astype(_BF)
        z = yb * (0.5 + 0.5 * jax.lax.erf(yb * _INV_SQRT2))
        y = _mm(z, c2) + c2b
        enc = _ln(enc + y, n2g, n2b, cmean)

    enc = enc * gs_ref[...] + bnb_ref[...]
    hwt = hwt_ref[...]
    decs = []
    for g in range(NG):
        acc = jnp.broadcast_to(hb_ref[...], (SG, PRED))
        for p in range(PT):
            r0 = g * PG + p * SG
            acc = acc + _mm(enc[r0:r0 + SG, :], hwt[D * p:D * (p + 1), :])
        decs.append(acc)
    dec = jnp.concatenate(decs, axis=0)
    out_ref[...] = dec * std + mu


def kernel(x_enc, x_mark_enc, x_dec, x_mark_dec, params):
    Bn, Ln, Cn = x_enc.shape
    N = Bn * Cn
    xt = jnp.transpose(x_enc, (0, 2, 1)).reshape(N, Ln)

    pp = params

    def bf_t(w):
        return jnp.transpose(w).astype(_BF)

    def row(b):
        return b[None, :].astype(_F32)

    fixed = [
        jnp.transpose(pp['cls_w1']).astype(_F32), row(pp['cls_b1']),
        jnp.transpose(pp['cls_w2']).astype(_F32), row(pp['cls_b2']),
        jnp.transpose(pp['emb_w0']).astype(_F32),
        jnp.transpose(pp['emb_w1']).astype(_F32),
        jnp.asarray(_pos_emb_const()),
    ]
    idx = jnp.arange(PG, dtype=jnp.int32)
    same = (idx[:, None] % SG) == (idx[None, :] % SG)
    mask = jnp.where(same, 1.0, 0.0).astype(_BF)
    fixed.append(mask)
    cmean = jnp.full((D, 1), 1.0 / D, _BF)
    fixed.append(cmean)
    hid = jnp.arange(D, dtype=jnp.int32) // E
    blk8 = (hid[None, :] == jnp.arange(H, dtype=jnp.int32)[:, None]
            ).astype(_BF)
    fixed.append(blk8)

    layer_ws = []
    for lp in pp['layers']:
        layer_ws += [
            bf_t(lp['wq'] * 0.125), row(lp['bq'] * 0.125),
            bf_t(lp['wk']), row(lp['bk']),
            bf_t(lp['wv']), row(lp['bv']), bf_t(lp['wo']), row(lp['bo']),
            bf_t(lp['c1w']), row(lp['c1b']), bf_t(lp['c2w']), row(lp['c2b']),
            row(lp['n1g']), row(lp['n1b']), row(lp['n2g']), row(lp['n2b']),
        ]

    gscale = row(pp['bn_g'] / math.sqrt(1.0 + 1e-5))
    bn_b = row(pp['bn_b'])
    hwt = (pp['head_w'].reshape(PRED, D, PT).transpose(2, 1, 0)
           .reshape(PT * D, PRED).astype(_BF))
    head_b = row(pp['head_b'])
    tail = [gscale, bn_b, hwt, head_b]

    n_layer_args = len(layer_ws)

    def body(*refs):
        x_ref = refs[0]
        f = refs[1:11]
        lw = refs[11:11 + n_layer_args]
        gs, bnb, hwt_r, hb = refs[11 + n_layer_args:11 + n_layer_args + 4]
        out_ref = refs[-1]
        _fused_kernel(x_ref, *f, lw, gs, bnb, hwt_r, hb, out_ref)

    grid = (N // ROWS,)

    def full(a):
        nd = a.ndim
        return pl.BlockSpec(a.shape, lambda i, _nd=nd: (0,) * _nd)

    in_specs = [pl.BlockSpec((ROWS, Ln), lambda i: (i, 0))]
    in_specs += [full(a) for a in fixed + layer_ws + tail]

    out = pl.pallas_call(
        body,
        grid=grid,
        in_specs=in_specs,
        out_specs=pl.BlockSpec((ROWS, PRED), lambda i: (i, 0)),
        out_shape=jax.ShapeDtypeStruct((N, PRED), _F32),
        compiler_params=pltpu.CompilerParams(
            dimension_semantics=("arbitrary",)),
    )(xt, *fixed, *layer_ws, *tail)

    return jnp.transpose(out.reshape(Bn, Cn, PRED), (0, 2, 1))


# speedup vs baseline: 1.1920x; 1.1920x over previous
import math

import jax
import jax.numpy as jnp
import numpy as np
from jax.experimental import pallas as pl
from jax.experimental.pallas import tpu as pltpu

L = 96
D = 512
H = 8
E = 64
DFF = 2048
NLAYERS = 2
PRED = 96
PT = 12
REG = 6
ROWS = 64
NG = 4
SG = ROWS // NG
PG = SG * PT
P = ROWS * PT
_INV_SQRT2 = 1.0 / math.sqrt(2.0)
_BF = jnp.bfloat16
_F32 = jnp.float32


def _pos_emb_const():
    position = np.arange(PT, dtype=np.float32)[:, None]
    div = np.exp(np.arange(0, D, 2, dtype=np.float32) * -(math.log(10000.0) / D))
    pe = np.zeros((PT, D), dtype=np.float32)
    pe[:, 0::2] = np.sin(position * div)
    pe[:, 1::2] = np.cos(position * div)
    return pe


def _mm(a, b):
    return jax.lax.dot_general(
        a.astype(_BF), b,
        (((1,), (0,)), ((), ())), preferred_element_type=_F32)


def _mm32(a, b):
    return jax.lax.dot_general(
        a, b, (((1,), (0,)), ((), ())), preferred_element_type=_F32)


def _mm_nt(a, b):
    return jax.lax.dot_general(
        a, b, (((1,), (1,)), ((), ())), preferred_element_type=_F32)


def _ln(x, g, b, cmean):
    mu = _mm32(x, cmean)
    m2 = _mm32(x * x, cmean)
    var = m2 - mu * mu
    return (x - mu) * jax.lax.rsqrt(var + 1e-5) * g + b


def _fused_kernel(x_ref, w1t_ref, b1_ref, w2t_ref, b2_ref, e0t_ref, e1t_ref,
                  pe_ref, mask_ref, cmean_ref, blk8_ref, lw_refs, gs_ref,
                  bnb_ref, hwt_ref, hb_ref, out_ref):
    x = x_ref[...]
    mu = jnp.mean(x, axis=1, keepdims=True)
    xc = x - mu
    var = jnp.mean(xc * xc, axis=1, keepdims=True)
    std = jnp.sqrt(var + 1e-5)
    xn = xc / std

    w1t = w1t_ref[...]
    w2t = w2t_ref[...]
    e0t = e0t_ref[...]
    e1t = e1t_ref[...]
    b1 = b1_ref[...]
    b2 = b2_ref[...]
    pe = pe_ref[...]

    tokens = [None] * (NG * PT)
    for r in range(REG):
        reg = xn[:, 16 * r:16 * r + 16]
        h1 = jax.nn.relu(_mm32(reg, w1t) + b1)
        lg = _mm32(h1, w2t) + b2
        sel = (lg[:, 0:1] >= lg[:, 1:2]).astype(_F32)
        e1 = _mm32(reg, e1t)
        for j in range(2):
            p = 2 * r + j
            pat0 = _mm32(xn[:, 8 * p:8 * p + 8], e0t)
            tok = e1 + sel * (pat0 - e1) + pe[p:p + 1, :]
            for g in range(NG):
                tokens[g * PT + p] = tok[g * SG:(g + 1) * SG]
    enc = jnp.concatenate(tokens, axis=0)

    mask = mask_ref[...]
    cmean = cmean_ref[...]
    ones_col = jnp.ones((PG, 1), _BF)
    for lyr in range(NLAYERS):
        (wq, bq, wk, bk, wv, bv, wo, bo,
         c1, c1b, c2, c2b, n1g, n1b, n2g, n2b) = [
            ref[...] for ref in lw_refs[16 * lyr:16 * (lyr + 1)]]
        xb = enc.astype(_BF)
        q = (_mm(xb, wq) + bq).astype(_BF)
        k = (_mm(xb, wk) + bk).astype(_BF)
        v = (_mm(xb, wv) + bv).astype(_BF)
        parts = []
        for g in range(NG):
            r0 = g * PG
            heads = []
            for h in range(H):
                qh = q[r0:r0 + PG, E * h:E * h + E]
                kh = k[r0:r0 + PG, E * h:E * h + E]
                s = _mm_nt(qh, kh)
                a = jnp.exp(s).astype(_BF) * mask
                vh = jnp.concatenate(
                    [v[r0:r0 + PG, E * h:E * h + E], ones_col], axis=1)
                o = _mm(a, vh)
                heads.append(o[:, :E] * (1.0 / o[:, E:E + 1]))
            parts.append(jnp.concatenate(heads, axis=1))
        ao = jnp.concatenate(parts, axis=0)
        ao = _mm(ao, wo) + bo
        enc = enc + ao
        enc = _ln(enc, n1g, n1b, cmean)
        y = _mm(enc, c1) + c1b
        yb = y.astype(_BF)
        z = yb * (0.5 + 0.5 * jax.lax.erf(yb * _INV_SQRT2))
        y = _mm(z, c2) + c2b
        enc = _ln(enc + y, n2g, n2b, cmean)

    enc = enc * gs_ref[...] + bnb_ref[...]
    hwt = hwt_ref[...]
    decs = []
    for g in range(NG):
        acc = jnp.broadcast_to(hb_ref[...], (SG, PRED))
        for p in range(PT):
            r0 = g * PG + p * SG
            acc = acc + _mm(enc[r0:r0 + SG, :], hwt[D * p:D * (p + 1), :])
        decs.append(acc)
    dec = jnp.concatenate(decs, axis=0)
    out_ref[...] = dec * std + mu


def kernel(x_enc, x_mark_enc, x_dec, x_mark_dec, params):
    Bn, Ln, Cn = x_enc.shape
    N = Bn * Cn
    xt = jnp.transpose(x_enc, (0, 2, 1)).reshape(N, Ln)

    pp = params

    def bf_t(w):
        return jnp.transpose(w).astype(_BF)

    def row(b):
        return b[None, :].astype(_F32)

    fixed = [
        jnp.transpose(pp['cls_w1']).astype(_F32), row(pp['cls_b1']),
        jnp.transpose(pp['cls_w2']).astype(_F32), row(pp['cls_b2']),
        jnp.transpose(pp['emb_w0']).astype(_F32),
        jnp.transpose(pp['emb_w1']).astype(_F32),
        jnp.asarray(_pos_emb_const()),
    ]
    idx = jnp.arange(PG, dtype=jnp.int32)
    same = (idx[:, None] % SG) == (idx[None, :] % SG)
    mask = jnp.where(same, 1.0, 0.0).astype(_BF)
    fixed.append(mask)
    cmean = jnp.full((D, 1), 1.0 / D, _F32)
    fixed.append(cmean)
    hid = jnp.arange(D, dtype=jnp.int32) // E
    blk8 = (hid[None, :] == jnp.arange(H, dtype=jnp.int32)[:, None]
            ).astype(_BF)
    fixed.append(blk8)

    layer_ws = []
    for lp in pp['layers']:
        layer_ws += [
            bf_t(lp['wq'] * 0.125), row(lp['bq'] * 0.125),
            bf_t(lp['wk']), row(lp['bk']),
            bf_t(lp['wv']), row(lp['bv']), bf_t(lp['wo']), row(lp['bo']),
            bf_t(lp['c1w']), row(lp['c1b']), bf_t(lp['c2w']), row(lp['c2b']),
            row(lp['n1g']), row(lp['n1b']), row(lp['n2g']), row(lp['n2b']),
        ]

    gscale = row(pp['bn_g'] / math.sqrt(1.0 + 1e-5))
    bn_b = row(pp['bn_b'])
    hwt = (pp['head_w'].reshape(PRED, D, PT).transpose(2, 1, 0)
           .reshape(PT * D, PRED).astype(_BF))
    head_b = row(pp['head_b'])
    tail = [gscale, bn_b, hwt, head_b]

    n_layer_args = len(layer_ws)

    def body(*refs):
        x_ref = refs[0]
        f = refs[1:11]
        lw = refs[11:11 + n_layer_args]
        gs, bnb, hwt_r, hb = refs[11 + n_layer_args:11 + n_layer_args + 4]
        out_ref = refs[-1]
        _fused_kernel(x_ref, *f, lw, gs, bnb, hwt_r, hb, out_ref)

    grid = (N // ROWS,)

    def full(a):
        nd = a.ndim
        return pl.BlockSpec(a.shape, lambda i, _nd=nd: (0,) * _nd)

    in_specs = [pl.BlockSpec((ROWS, Ln), lambda i: (i, 0))]
    in_specs += [full(a) for a in fixed + layer_ws + tail]

    out = pl.pallas_call(
        body,
        grid=grid,
        in_specs=in_specs,
        out_specs=pl.BlockSpec((ROWS, PRED), lambda i: (i, 0)),
        out_shape=jax.ShapeDtypeStruct((N, PRED), _F32),
        compiler_params=pltpu.CompilerParams(
            dimension_semantics=("arbitrary",)),
    )(xt, *fixed, *layer_ws, *tail)

    return jnp.transpose(out.reshape(Bn, Cn, PRED), (0, 2, 1))
